# Initial kernel scaffold; baseline (speedup 1.0000x reference)
import jax
import jax.numpy as jnp
from jax.experimental import pallas as pl


def kernel(inputs, table, gamma, beta, dimensions):
    raise NotImplementedError("write your pallas kernel here")


# speedup vs baseline: 2.3732x; 2.3732x over previous
import jax
import jax.numpy as jnp
from jax.experimental import pallas as pl

EPS = 1e-6


def _fused_ln_add(x_ref, tab_ref, g_ref, b_ref, o_ref):
    tab = tab_ref[...]
    mean = jnp.mean(tab, axis=-1, keepdims=True)
    c = tab - mean
    var = jnp.mean(c * c, axis=-1, keepdims=True)
    norm = c * jax.lax.rsqrt(var + EPS) * g_ref[...] + b_ref[...]
    o_ref[...] = x_ref[...] + norm[None, :, :]


def kernel(inputs, table, gamma, beta, dimensions):
    B, T, H = inputs.shape
    Tb = 256
    g2 = gamma.reshape(1, H)
    b2 = beta.reshape(1, H)
    return pl.pallas_call(
        _fused_ln_add,
        grid=(T // Tb,),
        in_specs=[
            pl.BlockSpec((B, Tb, H), lambda i: (0, i, 0)),
            pl.BlockSpec((Tb, H), lambda i: (i, 0)),
            pl.BlockSpec((1, H), lambda i: (0, 0)),
            pl.BlockSpec((1, H), lambda i: (0, 0)),
        ],
        out_specs=pl.BlockSpec((B, Tb, H), lambda i: (0, i, 0)),
        out_shape=jax.ShapeDtypeStruct((B, T, H), inputs.dtype),
    )(inputs, table, g2, b2)
